# Initial kernel scaffold; baseline (speedup 1.0000x reference)
import functools

import jax
import jax.numpy as jnp
from jax import lax
from jax.experimental import pallas as pl
from jax.experimental.pallas import tpu as pltpu
from jax.experimental.pallas import tpu_sc as plsc

V, D, B, K = 1000000, 64, 16384, 20
NC, NS = 2, 16
NW = NC * NS
BPW = B // NW
C = 32
NCHUNK = BPW // C
G = 128
NEG_PER_CHUNK = C * K
NG = NEG_PER_CHUNK // G
L = 16


def _sc_dots(pu_hbm, pv_hbm, ng_hbm, ut_hbm, vt_hbm,
             pd_hbm, nd_hbm,
             pu_idx, pv_idx, ng_idx, u_rows, v_rows, n_rows, pd, nd, sem):
    w = lax.axis_index("s") * NC + lax.axis_index("c")
    pltpu.sync_copy(pu_hbm.at[w], pu_idx)
    pltpu.sync_copy(pv_hbm.at[w], pv_idx)
    pltpu.sync_copy(ng_hbm.at[w], ng_idx)

    def chunk_body(c, _):
        cps = [
            pltpu.async_copy(ut_hbm.at[pu_idx.at[c]], u_rows, sem),
            pltpu.async_copy(vt_hbm.at[pv_idx.at[c]], v_rows, sem),
        ]
        for g in range(NG):
            cps.append(pltpu.async_copy(
                vt_hbm.at[ng_idx.at[c * NG + g]],
                n_rows.at[pl.ds(g * G, G)], sem))
        for cp in cps:
            cp.wait()

        def b_body(b, _):
            u0 = u_rows[b, pl.ds(0, L)]
            u1 = u_rows[b, pl.ds(L, L)]
            u2 = u_rows[b, pl.ds(2 * L, L)]
            u3 = u_rows[b, pl.ds(3 * L, L)]
            acc = (u0 * v_rows[b, pl.ds(0, L)]
                   + u1 * v_rows[b, pl.ds(L, L)]
                   + u2 * v_rows[b, pl.ds(2 * L, L)]
                   + u3 * v_rows[b, pl.ds(3 * L, L)])
            pd[c, b] = jnp.sum(acc)
            r0 = b * K
            for k in range(K):
                nacc = (u0 * n_rows[r0 + k, pl.ds(0, L)]
                        + u1 * n_rows[r0 + k, pl.ds(L, L)]
                        + u2 * n_rows[r0 + k, pl.ds(2 * L, L)]
                        + u3 * n_rows[r0 + k, pl.ds(3 * L, L)])
                nd[c, r0 + k] = jnp.sum(nacc)
            return 0

        lax.fori_loop(0, C, b_body, 0)
        return 0

    lax.fori_loop(0, NCHUNK, chunk_body, 0)
    pltpu.sync_copy(pd, pd_hbm.at[w])
    pltpu.sync_copy(nd, nd_hbm.at[w])


def _tail_body(pd_ref, nd_ref, o_ref):
    pos = pd_ref[...]
    neg = nd_ref[...]
    s_pos = jnp.sum(jax.nn.log_sigmoid(pos))
    s_neg = jnp.sum(jax.nn.log_sigmoid(-neg))
    o_ref[0, 0] = -(s_pos + s_neg)


def kernel(pos_u, pos_v, neg_v, u_table, v_table):
    pu = pos_u.astype(jnp.int32).reshape(NW, NCHUNK, C)
    pv = pos_v.astype(jnp.int32).reshape(NW, NCHUNK, C)
    ng = neg_v.astype(jnp.int32).reshape(NW, NCHUNK * NG, G)

    mesh = plsc.VectorSubcoreMesh(core_axis_name="c", subcore_axis_name="s")
    pd, nd = pl.kernel(
        _sc_dots,
        out_type=(
            jax.ShapeDtypeStruct((NW, NCHUNK, C), jnp.float32),
            jax.ShapeDtypeStruct((NW, NCHUNK, NEG_PER_CHUNK), jnp.float32),
        ),
        mesh=mesh,
        scratch_types=[
            pltpu.VMEM((NCHUNK, C), jnp.int32),
            pltpu.VMEM((NCHUNK, C), jnp.int32),
            pltpu.VMEM((NCHUNK * NG, G), jnp.int32),
            pltpu.VMEM((C, D), jnp.float32),
            pltpu.VMEM((C, D), jnp.float32),
            pltpu.VMEM((NEG_PER_CHUNK, D), jnp.float32),
            pltpu.VMEM((NCHUNK, C), jnp.float32),
            pltpu.VMEM((NCHUNK, NEG_PER_CHUNK), jnp.float32),
            pltpu.SemaphoreType.DMA,
        ],
    )(pu, pv, ng, u_table, v_table)

    out = pl.pallas_call(
        _tail_body,
        out_shape=jax.ShapeDtypeStruct((1, 1), jnp.float32),
    )(pd.reshape(B // 128, 128), nd.reshape(B * K // 128, 128))
    return out[0, 0]


# speedup vs baseline: 4.9400x; 4.9400x over previous
import functools

import jax
import jax.numpy as jnp
from jax import lax
from jax.experimental import pallas as pl
from jax.experimental.pallas import tpu as pltpu
from jax.experimental.pallas import tpu_sc as plsc

V, D, B, K = 1000000, 64, 16384, 20
NC, NS = 2, 16
NW = NC * NS
BPW = B // NW
C = 32
NCHUNK = BPW // C
G = 128
NEG_PER_CHUNK = C * K
NG = NEG_PER_CHUNK // G
L = 16
QB = 4


def _sc_dots(pu_hbm, pv_hbm, ng_hbm, ut_hbm, vt_hbm,
             pd_hbm, nd_hbm,
             pu_idx, pv_idx, ng_idx, u_rows, v_rows, n_rows, acc_scr,
             pos_scr, pd, nd, sem):
    w = lax.axis_index("s") * NC + lax.axis_index("c")
    pltpu.sync_copy(pu_hbm.at[w], pu_idx)
    pltpu.sync_copy(pv_hbm.at[w], pv_idx)
    pltpu.sync_copy(ng_hbm.at[w], ng_idx)

    def chunk_body(c, _):
        cps = [
            pltpu.async_copy(ut_hbm.at[pu_idx.at[c]], u_rows, sem),
            pltpu.async_copy(vt_hbm.at[pv_idx.at[c]], v_rows, sem),
        ]
        for g in range(NG):
            cps.append(pltpu.async_copy(
                vt_hbm.at[ng_idx.at[c * NG + g]],
                n_rows.at[pl.ds(g * G, G)], sem))
        for cp in cps:
            cp.wait()

        lanes = lax.iota(jnp.int32, L)

        def q_body(q, _):
            for bl in range(QB):
                b = q * QB + bl
                u0 = u_rows[b, pl.ds(0, L)]
                u1 = u_rows[b, pl.ds(L, L)]
                u2 = u_rows[b, pl.ds(2 * L, L)]
                u3 = u_rows[b, pl.ds(3 * L, L)]
                pacc = (u0 * v_rows[b, pl.ds(0, L)]
                        + u1 * v_rows[b, pl.ds(L, L)]
                        + u2 * v_rows[b, pl.ds(2 * L, L)]
                        + u3 * v_rows[b, pl.ds(3 * L, L)])
                pos_scr[pl.ds(b * L, L)] = pacc
                r0 = b * K
                for k in range(K):
                    nacc = (u0 * n_rows[r0 + k, pl.ds(0, L)]
                            + u1 * n_rows[r0 + k, pl.ds(L, L)]
                            + u2 * n_rows[r0 + k, pl.ds(2 * L, L)]
                            + u3 * n_rows[r0 + k, pl.ds(3 * L, L)])
                    acc_scr[pl.ds((bl * K + k) * L, L)] = nacc
            for m in range(QB * K // L):
                base = m * L * L
                red = plsc.load_gather(acc_scr, [lanes * L + base])
                for j in range(1, L):
                    red = red + plsc.load_gather(acc_scr, [lanes * L + (base + j)])
                nd[c, pl.ds(q * QB * K + m * L, L)] = red
            return 0

        lax.fori_loop(0, C // QB, q_body, 0)

        for m in range(C // L):
            base = m * L * L
            red = plsc.load_gather(pos_scr, [lanes * L + base])
            for j in range(1, L):
                red = red + plsc.load_gather(pos_scr, [lanes * L + (base + j)])
            pd[c, pl.ds(m * L, L)] = red
        return 0

    lax.fori_loop(0, NCHUNK, chunk_body, 0)
    pltpu.sync_copy(pd, pd_hbm.at[w])
    pltpu.sync_copy(nd, nd_hbm.at[w])


def _tail_body(pd_ref, nd_ref, o_ref):
    pos = pd_ref[...]
    neg = nd_ref[...]
    s_pos = jnp.sum(jax.nn.log_sigmoid(pos))
    s_neg = jnp.sum(jax.nn.log_sigmoid(-neg))
    o_ref[0, 0] = -(s_pos + s_neg)


def kernel(pos_u, pos_v, neg_v, u_table, v_table):
    pu = pos_u.astype(jnp.int32).reshape(NW, NCHUNK, C)
    pv = pos_v.astype(jnp.int32).reshape(NW, NCHUNK, C)
    ng = neg_v.astype(jnp.int32).reshape(NW, NCHUNK * NG, G)

    mesh = plsc.VectorSubcoreMesh(core_axis_name="c", subcore_axis_name="s")
    pd, nd = pl.kernel(
        _sc_dots,
        out_type=(
            jax.ShapeDtypeStruct((NW, NCHUNK, C), jnp.float32),
            jax.ShapeDtypeStruct((NW, NCHUNK, NEG_PER_CHUNK), jnp.float32),
        ),
        mesh=mesh,
        compiler_params=pltpu.CompilerParams(
            needs_layout_passes=False, use_tc_tiling_on_sc=False),
        scratch_types=[
            pltpu.VMEM((NCHUNK, C), jnp.int32),
            pltpu.VMEM((NCHUNK, C), jnp.int32),
            pltpu.VMEM((NCHUNK * NG, G), jnp.int32),
            pltpu.VMEM((C, D), jnp.float32),
            pltpu.VMEM((C, D), jnp.float32),
            pltpu.VMEM((NEG_PER_CHUNK, D), jnp.float32),
            pltpu.VMEM((QB * K * L,), jnp.float32),
            pltpu.VMEM((C * L,), jnp.float32),
            pltpu.VMEM((NCHUNK, C), jnp.float32),
            pltpu.VMEM((NCHUNK, NEG_PER_CHUNK), jnp.float32),
            pltpu.SemaphoreType.DMA,
        ],
    )(pu, pv, ng, u_table, v_table)

    out = pl.pallas_call(
        _tail_body,
        out_shape=jax.ShapeDtypeStruct((1, 1), jnp.float32),
        out_specs=pl.BlockSpec(memory_space=pltpu.SMEM),
    )(pd.reshape(B // 128, 128), nd.reshape(B * K // 128, 128))
    return out[0, 0]


# speedup vs baseline: 8.2595x; 1.6719x over previous
import functools

import jax
import jax.numpy as jnp
from jax import lax
from jax.experimental import pallas as pl
from jax.experimental.pallas import tpu as pltpu
from jax.experimental.pallas import tpu_sc as plsc

V, D, B, K = 1000000, 64, 16384, 20
DP = 128
TRB = 4096
NC, NS = 2, 16
NW = NC * NS
BPW = B // NW
C = 32
NCHUNK = BPW // C
G = 128
NEG_PER_CHUNK = C * K
NG = NEG_PER_CHUNK // G
L = 16
QB = 4


def _detranspose_body(ut_ref, vt_ref, up_ref, vp_ref):
    up_ref[:, 0:D] = ut_ref[...].T
    vp_ref[:, 0:D] = vt_ref[...].T


def _detranspose(u_table, v_table):
    grid = (V + TRB - 1) // TRB
    return pl.pallas_call(
        _detranspose_body,
        grid=(grid,),
        in_specs=[
            pl.BlockSpec((D, TRB), lambda j: (0, j)),
            pl.BlockSpec((D, TRB), lambda j: (0, j)),
        ],
        out_specs=[
            pl.BlockSpec((TRB, DP), lambda j: (j, 0)),
            pl.BlockSpec((TRB, DP), lambda j: (j, 0)),
        ],
        out_shape=(
            jax.ShapeDtypeStruct((V, DP), jnp.float32),
            jax.ShapeDtypeStruct((V, DP), jnp.float32),
        ),
    )(u_table.T, v_table.T)


def _sc_dots(pu_hbm, pv_hbm, ng_hbm, ut_hbm, vt_hbm,
             pd_hbm, nd_hbm,
             pu_idx, pv_idx, ng_idx, u_rows, v_rows, n_rows, acc_scr,
             pos_scr, pd, nd, sem):
    w = lax.axis_index("s") * NC + lax.axis_index("c")
    pltpu.sync_copy(pu_hbm.at[w], pu_idx)
    pltpu.sync_copy(pv_hbm.at[w], pv_idx)
    pltpu.sync_copy(ng_hbm.at[w], ng_idx)

    def chunk_body(c, _):
        cps = [
            pltpu.async_copy(ut_hbm.at[pu_idx.at[c]], u_rows, sem),
            pltpu.async_copy(vt_hbm.at[pv_idx.at[c]], v_rows, sem),
        ]
        for g in range(NG):
            cps.append(pltpu.async_copy(
                vt_hbm.at[ng_idx.at[c * NG + g]],
                n_rows.at[pl.ds(g * G, G)], sem))
        for cp in cps:
            cp.wait()

        lanes = lax.iota(jnp.int32, L)

        def q_body(q, _):
            for bl in range(QB):
                b = q * QB + bl
                u0 = u_rows[b, pl.ds(0, L)]
                u1 = u_rows[b, pl.ds(L, L)]
                u2 = u_rows[b, pl.ds(2 * L, L)]
                u3 = u_rows[b, pl.ds(3 * L, L)]
                pacc = (u0 * v_rows[b, pl.ds(0, L)]
                        + u1 * v_rows[b, pl.ds(L, L)]
                        + u2 * v_rows[b, pl.ds(2 * L, L)]
                        + u3 * v_rows[b, pl.ds(3 * L, L)])
                pos_scr[pl.ds(b * L, L)] = pacc
                r0 = b * K
                for k in range(K):
                    nacc = (u0 * n_rows[r0 + k, pl.ds(0, L)]
                            + u1 * n_rows[r0 + k, pl.ds(L, L)]
                            + u2 * n_rows[r0 + k, pl.ds(2 * L, L)]
                            + u3 * n_rows[r0 + k, pl.ds(3 * L, L)])
                    acc_scr[pl.ds((bl * K + k) * L, L)] = nacc
            for m in range(QB * K // L):
                base = m * L * L
                red = plsc.load_gather(acc_scr, [lanes * L + base])
                for j in range(1, L):
                    red = red + plsc.load_gather(acc_scr, [lanes * L + (base + j)])
                nd[c, pl.ds(q * QB * K + m * L, L)] = red
            return 0

        lax.fori_loop(0, C // QB, q_body, 0)

        for m in range(C // L):
            base = m * L * L
            red = plsc.load_gather(pos_scr, [lanes * L + base])
            for j in range(1, L):
                red = red + plsc.load_gather(pos_scr, [lanes * L + (base + j)])
            pd[c, pl.ds(m * L, L)] = red
        return 0

    lax.fori_loop(0, NCHUNK, chunk_body, 0)
    pltpu.sync_copy(pd, pd_hbm.at[w])
    pltpu.sync_copy(nd, nd_hbm.at[w])


def _tail_body(pd_ref, nd_ref, o_ref):
    pos = pd_ref[...]
    neg = nd_ref[...]
    s_pos = jnp.sum(jax.nn.log_sigmoid(pos))
    s_neg = jnp.sum(jax.nn.log_sigmoid(-neg))
    o_ref[0, 0] = -(s_pos + s_neg)


def kernel(pos_u, pos_v, neg_v, u_table, v_table):
    pu = pos_u.astype(jnp.int32).reshape(NW, NCHUNK, C)
    pv = pos_v.astype(jnp.int32).reshape(NW, NCHUNK, C)
    ng = neg_v.astype(jnp.int32).reshape(NW, NCHUNK * NG, G)

    ut_p, vt_p = _detranspose(u_table, v_table)

    mesh = plsc.VectorSubcoreMesh(core_axis_name="c", subcore_axis_name="s")
    pd, nd = pl.kernel(
        _sc_dots,
        out_type=(
            jax.ShapeDtypeStruct((NW, NCHUNK, C), jnp.float32),
            jax.ShapeDtypeStruct((NW, NCHUNK, NEG_PER_CHUNK), jnp.float32),
        ),
        mesh=mesh,
        compiler_params=pltpu.CompilerParams(
            needs_layout_passes=False, use_tc_tiling_on_sc=False),
        scratch_types=[
            pltpu.VMEM((NCHUNK, C), jnp.int32),
            pltpu.VMEM((NCHUNK, C), jnp.int32),
            pltpu.VMEM((NCHUNK * NG, G), jnp.int32),
            pltpu.VMEM((C, DP), jnp.float32),
            pltpu.VMEM((C, DP), jnp.float32),
            pltpu.VMEM((NEG_PER_CHUNK, DP), jnp.float32),
            pltpu.VMEM((QB * K * L,), jnp.float32),
            pltpu.VMEM((C * L,), jnp.float32),
            pltpu.VMEM((NCHUNK, C), jnp.float32),
            pltpu.VMEM((NCHUNK, NEG_PER_CHUNK), jnp.float32),
            pltpu.SemaphoreType.DMA,
        ],
    )(pu, pv, ng, ut_p, vt_p)

    out = pl.pallas_call(
        _tail_body,
        out_shape=jax.ShapeDtypeStruct((1, 1), jnp.float32),
        out_specs=pl.BlockSpec(memory_space=pltpu.SMEM),
    )(pd.reshape(B // 128, 128), nd.reshape(B * K // 128, 128))
    return out[0, 0]
